# baseline (speedup 1.0000x reference)
import functools

import jax
import jax.numpy as jnp
from jax import lax
from jax.experimental import pallas as pl
from jax.experimental.pallas import tpu as pltpu
from jax.experimental.pallas import tpu_sc as plsc

_W = 16
_PB = 256


def _sc_gather(table, idx_i, idx_j, ppad, f):
    mesh = plsc.VectorSubcoreMesh(core_axis_name="c", subcore_axis_name="s")
    n_workers = mesh.num_cores * mesh.num_subcores
    bpw = ppad // n_workers
    nch = bpw // _W

    out_t = (
        jax.ShapeDtypeStruct((ppad, f), jnp.float32),
        jax.ShapeDtypeStruct((ppad, f), jnp.float32),
    )

    @functools.partial(
        pl.kernel, out_type=out_t, mesh=mesh,
        scratch_types=[
            pltpu.VMEM((bpw,), jnp.int32),
            pltpu.VMEM((bpw,), jnp.int32),
            pltpu.VMEM((_W, f), jnp.float32),
            pltpu.VMEM((_W, f), jnp.float32),
            pltpu.VMEM((_W, f), jnp.float32),
            pltpu.VMEM((_W, f), jnp.float32),
            pltpu.SemaphoreType.DMA,
            pltpu.SemaphoreType.DMA,
            pltpu.SemaphoreType.DMA,
            pltpu.SemaphoreType.DMA,
        ])
    def gather_kernel(table_hbm, ii_hbm, ij_hbm, gi_hbm, gj_hbm,
                      ii_v, ij_v, bi0, bj0, bi1, bj1, si0, sj0, si1, sj1):
        wid = lax.axis_index("s") * mesh.num_cores + lax.axis_index("c")
        base = wid * bpw
        pltpu.sync_copy(ii_hbm.at[pl.ds(base, bpw)], ii_v)
        pltpu.sync_copy(ij_hbm.at[pl.ds(base, bpw)], ij_v)

        def fire(c, bi, bj, si, sj):
            pltpu.make_async_copy(
                table_hbm.at[ii_v.at[pl.ds(c * _W, _W)]], bi, si).start()
            pltpu.make_async_copy(
                table_hbm.at[ij_v.at[pl.ds(c * _W, _W)]], bj, sj).start()

        def drain(c, bi, bj, si, sj):
            pltpu.make_async_copy(
                table_hbm.at[ii_v.at[pl.ds(c * _W, _W)]], bi, si).wait()
            pltpu.make_async_copy(
                table_hbm.at[ij_v.at[pl.ds(c * _W, _W)]], bj, sj).wait()
            pltpu.sync_copy(bi, gi_hbm.at[pl.ds(base + c * _W, _W)])
            pltpu.sync_copy(bj, gj_hbm.at[pl.ds(base + c * _W, _W)])

        fire(0, bi0, bj0, si0, sj0)

        @pl.loop(0, nch, step=2)
        def _(c):
            fire(c + 1, bi1, bj1, si1, sj1)
            drain(c, bi0, bj0, si0, sj0)

            @pl.when(c + 2 < nch)
            def _():
                fire(c + 2, bi0, bj0, si0, sj0)

            drain(c + 1, bi1, bj1, si1, sj1)

    return gather_kernel(table, idx_i, idx_j)


def _tc_body(nbins, d, t_len, beta_col, times_ref, gi_ref, gj_ref,
             out_ref, abc_ref):
    gi = gi_ref[...]
    gj = gj_ref[...]
    lane = lax.broadcasted_iota(jnp.int32, gi.shape, 1)
    x = jnp.where(lane == beta_col, gi + gj, gi - gj)
    xt = jnp.transpose(x)

    inv_w = jnp.float32(1.0 / nbins)
    x0_row = nbins * d
    acc = xt[x0_row:x0_row + d, :]
    for m in range(nbins):
        dvm = xt[d * m:d * (m + 1), :]
        abc_ref[nbins + m:nbins + m + 1, :] = jnp.sum(
            acc * dvm, axis=0, keepdims=True)
        abc_ref[2 * nbins + m:2 * nbins + m + 1, :] = jnp.sum(
            dvm * dvm, axis=0, keepdims=True)
        acc = acc + dvm * inv_w
    b_all = abc_ref[nbins:2 * nbins, :]
    c_all = abc_ref[2 * nbins:3 * nbins, :]
    dx0 = xt[x0_row:x0_row + d, :]
    a0 = jnp.sum(dx0 * dx0, axis=0, keepdims=True)
    step = 2.0 * inv_w * b_all + (inv_w * inv_w) * c_all
    row = lax.broadcasted_iota(jnp.int32, (nbins, nbins), 0)
    col = lax.broadcasted_iota(jnp.int32, (nbins, nbins), 1)
    ltri = jnp.where(col < row, jnp.float32(1.0), jnp.float32(0.0))
    a_all = a0 + lax.dot_general(
        ltri, step, (((1,), (0,)), ((), ())),
        preferred_element_type=jnp.float32,
        precision=lax.Precision.HIGHEST)
    abc_ref[0:nbins, :] = a_all

    t = times_ref[...]
    mt = jnp.clip(jnp.floor(t * nbins), 0.0, nbins - 1.0)
    r = t - mt * inv_w
    lane2 = lax.broadcasted_iota(jnp.int32, (t_len, 3 * nbins), 1)
    binl = (lane2 % nbins).astype(jnp.float32)
    coef = jnp.where(lane2 < nbins, jnp.float32(1.0),
                     jnp.where(lane2 < 2 * nbins, 2.0 * r, r * r))
    sel = jnp.where(binl == mt, coef, jnp.float32(0.0))

    norm2 = lax.dot_general(
        sel, abc_ref[...], (((1,), (0,)), ((), ())),
        preferred_element_type=jnp.float32,
        precision=lax.Precision.HIGHEST)
    bsum = xt[beta_col:beta_col + 1, :]
    out_ref[...] = jnp.exp(bsum - norm2)


def _tc_compute(times2d, gi, gj, nbins, d, f, beta_col, ppad):
    t_len = times2d.shape[0]
    body = functools.partial(_tc_body, nbins, d, t_len, beta_col)
    return pl.pallas_call(
        body,
        grid=(ppad // _PB,),
        in_specs=[
            pl.BlockSpec((t_len, 1), lambda p: (0, 0)),
            pl.BlockSpec((_PB, f), lambda p: (p, 0)),
            pl.BlockSpec((_PB, f), lambda p: (p, 0)),
        ],
        out_specs=pl.BlockSpec((t_len, _PB), lambda p: (0, p)),
        out_shape=jax.ShapeDtypeStruct((t_len, ppad), jnp.float32),
        scratch_shapes=[pltpu.VMEM((3 * nbins, _PB), jnp.float32)],
    )(times2d, gi, gj)


def kernel(x0, v, beta, times_list, node_pairs):
    n, d = x0.shape
    nbins = v.shape[0]
    p = node_pairs.shape[1]

    beta_col = nbins * d + d
    f = ((beta_col + 1 + 127) // 128) * 128
    vt = jnp.transpose(v, (1, 0, 2)).reshape(n, nbins * d)
    table = jnp.concatenate(
        [vt, x0, beta[:, None],
         jnp.zeros((n, f - beta_col - 1), jnp.float32)], axis=1)

    nslice = 5
    align = nslice * max(_W * 64, _PB)
    ppad = ((p + align - 1) // align) * align
    idx = jnp.pad(node_pairs, ((0, 0), (0, ppad - p)))

    psl = ppad // nslice
    times2d = times_list[:, None]
    outs = []
    for s in range(nslice):
        ii = lax.dynamic_slice_in_dim(idx[0], s * psl, psl)
        ij = lax.dynamic_slice_in_dim(idx[1], s * psl, psl)
        gi, gj = _sc_gather(table, ii, ij, psl, f)
        outs.append(
            _tc_compute(times2d, gi, gj, nbins, d, f, beta_col, psl))
    out = jnp.concatenate(outs, axis=1)
    return out[:, :p]


# speedup vs baseline: 1.2531x; 1.2531x over previous
import functools

import jax
import jax.numpy as jnp
from jax import lax
from jax.experimental import pallas as pl
from jax.experimental.pallas import tpu as pltpu
from jax.experimental.pallas import tpu_sc as plsc

_W = 16
_PB = 256
_FXB = 128


def _sc_gather(tv, txb, idx_i, idx_j, ppad, fv):
    mesh = plsc.VectorSubcoreMesh(core_axis_name="c", subcore_axis_name="s")
    n_workers = mesh.num_cores * mesh.num_subcores
    bpw = ppad // n_workers
    nch = bpw // _W

    out_t = (
        jax.ShapeDtypeStruct((ppad, fv), jnp.float32),
        jax.ShapeDtypeStruct((ppad, fv), jnp.float32),
        jax.ShapeDtypeStruct((ppad, _FXB), jnp.float32),
        jax.ShapeDtypeStruct((ppad, _FXB), jnp.float32),
    )
    buf_t = [pltpu.VMEM((_W, fv), jnp.float32)] * 2 \
        + [pltpu.VMEM((_W, _FXB), jnp.float32)] * 2

    @functools.partial(
        pl.kernel, out_type=out_t, mesh=mesh,
        scratch_types=[
            pltpu.VMEM((bpw,), jnp.int32),
            pltpu.VMEM((bpw,), jnp.int32),
        ] + buf_t + buf_t + [pltpu.SemaphoreType.DMA] * 8)
    def gather_kernel(tv_hbm, txb_hbm, ii_hbm, ij_hbm,
                      gvi_hbm, gvj_hbm, gxi_hbm, gxj_hbm,
                      ii_v, ij_v,
                      bvi0, bvj0, bxi0, bxj0,
                      bvi1, bvj1, bxi1, bxj1,
                      svi0, svj0, sxi0, sxj0,
                      svi1, svj1, sxi1, sxj1):
        wid = lax.axis_index("s") * mesh.num_cores + lax.axis_index("c")
        base = wid * bpw
        pltpu.sync_copy(ii_hbm.at[pl.ds(base, bpw)], ii_v)
        pltpu.sync_copy(ij_hbm.at[pl.ds(base, bpw)], ij_v)

        def fire(c, bufs, sems):
            bvi, bvj, bxi, bxj = bufs
            svi, svj, sxi, sxj = sems
            ii_s = ii_v.at[pl.ds(c * _W, _W)]
            ij_s = ij_v.at[pl.ds(c * _W, _W)]
            pltpu.make_async_copy(tv_hbm.at[ii_s], bvi, svi).start()
            pltpu.make_async_copy(tv_hbm.at[ij_s], bvj, svj).start()
            pltpu.make_async_copy(txb_hbm.at[ii_s], bxi, sxi).start()
            pltpu.make_async_copy(txb_hbm.at[ij_s], bxj, sxj).start()

        def drain(c, bufs, sems):
            bvi, bvj, bxi, bxj = bufs
            svi, svj, sxi, sxj = sems
            ii_s = ii_v.at[pl.ds(c * _W, _W)]
            ij_s = ij_v.at[pl.ds(c * _W, _W)]
            pltpu.make_async_copy(tv_hbm.at[ii_s], bvi, svi).wait()
            pltpu.make_async_copy(tv_hbm.at[ij_s], bvj, svj).wait()
            pltpu.make_async_copy(txb_hbm.at[ii_s], bxi, sxi).wait()
            pltpu.make_async_copy(txb_hbm.at[ij_s], bxj, sxj).wait()
            dst = pl.ds(base + c * _W, _W)
            pltpu.sync_copy(bvi, gvi_hbm.at[dst])
            pltpu.sync_copy(bvj, gvj_hbm.at[dst])
            pltpu.sync_copy(bxi, gxi_hbm.at[dst])
            pltpu.sync_copy(bxj, gxj_hbm.at[dst])

        bufs0 = (bvi0, bvj0, bxi0, bxj0)
        bufs1 = (bvi1, bvj1, bxi1, bxj1)
        sems0 = (svi0, svj0, sxi0, sxj0)
        sems1 = (svi1, svj1, sxi1, sxj1)

        fire(0, bufs0, sems0)

        @pl.loop(0, nch, step=2)
        def _(c):
            fire(c + 1, bufs1, sems1)
            drain(c, bufs0, sems0)

            @pl.when(c + 2 < nch)
            def _():
                fire(c + 2, bufs0, sems0)

            drain(c + 1, bufs1, sems1)

    return gather_kernel(tv, txb, idx_i, idx_j)


def _tc_body(nbins, d, t_len, times_ref, gvi_ref, gvj_ref, gxi_ref, gxj_ref,
             out_ref, abc_ref):
    dvt = jnp.transpose(gvi_ref[...] - gvj_ref[...])
    gxi = gxi_ref[...]
    gxj = gxj_ref[...]
    lane = lax.broadcasted_iota(jnp.int32, gxi.shape, 1)
    xb = jnp.where(lane == d, gxi + gxj, gxi - gxj)
    xbt = jnp.transpose(xb)
    dx0 = xbt[0:d, :]
    bsum = xbt[d:d + 1, :]

    inv_w = jnp.float32(1.0 / nbins)
    acc = dx0
    for m in range(nbins):
        dvm = dvt[d * m:d * (m + 1), :]
        abc_ref[nbins + m:nbins + m + 1, :] = jnp.sum(
            acc * dvm, axis=0, keepdims=True)
        abc_ref[2 * nbins + m:2 * nbins + m + 1, :] = jnp.sum(
            dvm * dvm, axis=0, keepdims=True)
        acc = acc + dvm * inv_w
    b_all = abc_ref[nbins:2 * nbins, :]
    c_all = abc_ref[2 * nbins:3 * nbins, :]
    a0 = jnp.sum(dx0 * dx0, axis=0, keepdims=True)
    step = 2.0 * inv_w * b_all + (inv_w * inv_w) * c_all
    row = lax.broadcasted_iota(jnp.int32, (nbins, nbins), 0)
    col = lax.broadcasted_iota(jnp.int32, (nbins, nbins), 1)
    ltri = jnp.where(col < row, jnp.float32(1.0), jnp.float32(0.0))
    a_all = a0 + lax.dot_general(
        ltri, step, (((1,), (0,)), ((), ())),
        preferred_element_type=jnp.float32,
        precision=lax.Precision.HIGHEST)
    abc_ref[0:nbins, :] = a_all

    t = times_ref[...]
    mt = jnp.clip(jnp.floor(t * nbins), 0.0, nbins - 1.0)
    r = t - mt * inv_w
    lane2 = lax.broadcasted_iota(jnp.int32, (t_len, 3 * nbins), 1)
    binl = (lane2 % nbins).astype(jnp.float32)
    coef = jnp.where(lane2 < nbins, jnp.float32(1.0),
                     jnp.where(lane2 < 2 * nbins, 2.0 * r, r * r))
    sel = jnp.where(binl == mt, coef, jnp.float32(0.0))

    norm2 = lax.dot_general(
        sel, abc_ref[...], (((1,), (0,)), ((), ())),
        preferred_element_type=jnp.float32,
        precision=lax.Precision.HIGHEST)
    out_ref[...] = jnp.exp(bsum - norm2)


def _tc_compute(times2d, gvi, gvj, gxi, gxj, nbins, d, fv, ppad):
    t_len = times2d.shape[0]
    body = functools.partial(_tc_body, nbins, d, t_len)
    return pl.pallas_call(
        body,
        grid=(ppad // _PB,),
        in_specs=[
            pl.BlockSpec((t_len, 1), lambda p: (0, 0)),
            pl.BlockSpec((_PB, fv), lambda p: (p, 0)),
            pl.BlockSpec((_PB, fv), lambda p: (p, 0)),
            pl.BlockSpec((_PB, _FXB), lambda p: (p, 0)),
            pl.BlockSpec((_PB, _FXB), lambda p: (p, 0)),
        ],
        out_specs=pl.BlockSpec((t_len, _PB), lambda p: (0, p)),
        out_shape=jax.ShapeDtypeStruct((t_len, ppad), jnp.float32),
        scratch_shapes=[pltpu.VMEM((3 * nbins, _PB), jnp.float32)],
    )(times2d, gvi, gvj, gxi, gxj)


def kernel(x0, v, beta, times_list, node_pairs):
    n, d = x0.shape
    nbins = v.shape[0]
    p = node_pairs.shape[1]

    fv = nbins * d
    tv = jnp.transpose(v, (1, 0, 2)).reshape(n, fv)
    txb = jnp.concatenate(
        [x0, beta[:, None], jnp.zeros((n, _FXB - d - 1), jnp.float32)],
        axis=1)

    nslice = 5
    align = nslice * max(_W * 64, _PB)
    ppad = ((p + align - 1) // align) * align
    tail = (jnp.arange(ppad - p, dtype=jnp.int32) % n)[None, :]
    idx = jnp.concatenate(
        [node_pairs, jnp.concatenate([tail, tail], axis=0)], axis=1)

    psl = ppad // nslice
    times2d = times_list[:, None]
    outs = []
    for s in range(nslice):
        ii = lax.dynamic_slice_in_dim(idx[0], s * psl, psl)
        ij = lax.dynamic_slice_in_dim(idx[1], s * psl, psl)
        gvi, gvj, gxi, gxj = _sc_gather(tv, txb, ii, ij, psl, fv)
        outs.append(
            _tc_compute(times2d, gvi, gvj, gxi, gxj, nbins, d, fv, psl))
    out = jnp.concatenate(outs, axis=1)
    return out[:, :p]
